# baseline (speedup 1.0000x reference)
import functools

import jax
import jax.numpy as jnp
from jax import lax
from jax.experimental import pallas as pl
from jax.experimental.pallas import tpu as pltpu
from jax.experimental.pallas import tpu_sc as plsc

D_MODEL = 64
NIDX = 128
SUB = 4
CHUNK = NIDX * SUB


def _gather_impl(idx, emb_weight):
    N = idx.shape[0]
    info = plsc.get_sparse_core_info()
    NC, NS = info.num_cores, info.num_subcores
    NW = NC * NS
    per_w = N // NW
    n_chunks = per_w // CHUNK
    mesh = plsc.VectorSubcoreMesh(core_axis_name="c", subcore_axis_name="s")

    @functools.partial(
        pl.kernel,
        mesh=mesh,
        out_type=jax.ShapeDtypeStruct((N, D_MODEL), jnp.float32),
        scratch_types=[
            pltpu.VMEM((CHUNK,), jnp.int32),
            pltpu.VMEM((CHUNK, D_MODEL), jnp.float32),
            pltpu.SemaphoreType.DMA,
        ],
        compiler_params=pltpu.CompilerParams(use_tc_tiling_on_sc=False),
    )
    def k(idx_hbm, tab_hbm, out_hbm, idx_v, rows_v, sem):
        wid = lax.axis_index("s") * NC + lax.axis_index("c")
        base = wid * per_w

        def body(i, carry):
            off = base + i * CHUNK
            pltpu.sync_copy(idx_hbm.at[pl.ds(off, CHUNK)], idx_v)
            copies = []
            for j in range(SUB):
                copies.append(
                    pltpu.async_copy(
                        tab_hbm.at[idx_v.at[pl.ds(j * NIDX, NIDX)]],
                        rows_v.at[pl.ds(j * NIDX, NIDX)],
                        sem,
                    )
                )
            for c in copies:
                c.wait()
            pltpu.sync_copy(rows_v, out_hbm.at[pl.ds(off, CHUNK)])
            return carry

        lax.fori_loop(0, n_chunks, body, 0)

    return k(idx, emb_weight)


def kernel(x, emb_weight):
    B, L = x.shape
    N = B * L
    idx = x.reshape(N).astype(jnp.int32)
    out = _gather_impl(idx, emb_weight)
    return out.reshape(B, L, D_MODEL)


# speedup vs baseline: 1.0422x; 1.0422x over previous
import functools

import jax
import jax.numpy as jnp
from jax import lax
from jax.experimental import pallas as pl
from jax.experimental.pallas import tpu as pltpu
from jax.experimental.pallas import tpu_sc as plsc

D_MODEL = 64
NIDX = 128
SUB = 4
CHUNK = NIDX * SUB
NBUF = 2


def _gather_impl(idx, emb_weight):
    N = idx.shape[0]
    info = plsc.get_sparse_core_info()
    NC, NS = info.num_cores, info.num_subcores
    NW = NC * NS
    per_w = N // NW
    n_chunks = per_w // CHUNK
    assert n_chunks % NBUF == 0
    mesh = plsc.VectorSubcoreMesh(core_axis_name="c", subcore_axis_name="s")

    @functools.partial(
        pl.kernel,
        mesh=mesh,
        out_type=jax.ShapeDtypeStruct((N, D_MODEL), jnp.float32),
        scratch_types=[
            pltpu.VMEM((NBUF, CHUNK), jnp.int32),
            pltpu.VMEM((NBUF, CHUNK, D_MODEL), jnp.float32),
            pltpu.SemaphoreType.DMA,
            pltpu.SemaphoreType.DMA,
            pltpu.SemaphoreType.DMA,
        ],
        compiler_params=pltpu.CompilerParams(use_tc_tiling_on_sc=False),
    )
    def k(idx_hbm, tab_hbm, out_hbm, idx_v, rows_v, isem, gsem, osem):
        wid = lax.axis_index("s") * NC + lax.axis_index("c")
        base = wid * per_w

        def issue_idx(i, b):
            pltpu.async_copy(
                idx_hbm.at[pl.ds(base + i * CHUNK, CHUNK)], idx_v.at[b], isem)

        def wait_idx(b):
            pltpu.make_async_copy(
                idx_hbm.at[pl.ds(0, CHUNK)], idx_v.at[b], isem).wait()

        def issue_gathers(b):
            for j in range(SUB):
                pltpu.async_copy(
                    tab_hbm.at[idx_v.at[b].at[pl.ds(j * NIDX, NIDX)]],
                    rows_v.at[b].at[pl.ds(j * NIDX, NIDX)],
                    gsem,
                )

        def wait_gathers(b):
            for j in range(SUB):
                pltpu.make_async_copy(
                    out_hbm.at[pl.ds(j * NIDX, NIDX)],
                    rows_v.at[b].at[pl.ds(j * NIDX, NIDX)],
                    gsem,
                ).wait()

        def issue_out(i, b):
            pltpu.async_copy(
                rows_v.at[b], out_hbm.at[pl.ds(base + i * CHUNK, CHUNK)], osem)

        def wait_out(b):
            pltpu.make_async_copy(
                rows_v.at[b], out_hbm.at[pl.ds(0, CHUNK)], osem).wait()

        for b in range(NBUF):
            issue_idx(b, b)

        def pair_body(t, carry):
            for b in range(NBUF):
                i = t * NBUF + b
                @pl.when(i >= NBUF)
                def _():
                    wait_out(b)
                wait_idx(b)
                issue_gathers(b)
                wait_gathers(b)
                issue_out(i, b)

                @pl.when(i + NBUF < n_chunks)
                def _():
                    issue_idx(i + NBUF, b)
            return carry

        lax.fori_loop(0, n_chunks // NBUF, pair_body, 0)

        for b in range(NBUF):
            wait_out(b)

    return k(idx, emb_weight)


def kernel(x, emb_weight):
    B, L = x.shape
    N = B * L
    idx = x.reshape(N).astype(jnp.int32)
    out = _gather_impl(idx, emb_weight)
    return out.reshape(B, L, D_MODEL)


# speedup vs baseline: 1.0449x; 1.0025x over previous
import functools

import jax
import jax.numpy as jnp
from jax import lax
from jax.experimental import pallas as pl
from jax.experimental.pallas import tpu as pltpu
from jax.experimental.pallas import tpu_sc as plsc

D_MODEL = 64
R_CHUNK = 4
NBUF = 2
SPLITS = ((0, 80), (80, 80), (160, 40))


def _gather_impl(x, emb_weight):
    B, L = x.shape
    info = plsc.get_sparse_core_info()
    NC, NS = info.num_cores, info.num_subcores
    NW = NC * NS
    rows_per_w = B // NW
    n_chunks = rows_per_w // R_CHUNK
    assert n_chunks % NBUF == 0
    mesh = plsc.VectorSubcoreMesh(core_axis_name="c", subcore_axis_name="s")

    @functools.partial(
        pl.kernel,
        mesh=mesh,
        out_type=jax.ShapeDtypeStruct((B, L, D_MODEL), jnp.float32),
        scratch_types=[
            pltpu.VMEM((NBUF, R_CHUNK, L), jnp.int32),
            pltpu.VMEM((NBUF, R_CHUNK, L, D_MODEL), jnp.float32),
            pltpu.SemaphoreType.DMA,
            pltpu.SemaphoreType.DMA,
            pltpu.SemaphoreType.DMA,
        ],
        compiler_params=pltpu.CompilerParams(use_tc_tiling_on_sc=False),
    )
    def k(x_hbm, tab_hbm, out_hbm, idx_v, rows_v, isem, gsem, osem):
        wid = lax.axis_index("s") * NC + lax.axis_index("c")
        base = wid * rows_per_w

        def issue_idx(i, b):
            pltpu.async_copy(
                x_hbm.at[pl.ds(base + i * R_CHUNK, R_CHUNK)],
                idx_v.at[b], isem)

        def wait_idx(b):
            pltpu.make_async_copy(
                x_hbm.at[pl.ds(0, R_CHUNK)], idx_v.at[b], isem).wait()

        def issue_gathers(b):
            for r in range(R_CHUNK):
                for (o, ln) in SPLITS:
                    pltpu.async_copy(
                        tab_hbm.at[idx_v.at[b, r, pl.ds(o, ln)]],
                        rows_v.at[b, r, pl.ds(o, ln)],
                        gsem,
                    )

        def wait_gathers(b):
            for r in range(R_CHUNK):
                for (o, ln) in SPLITS:
                    pltpu.make_async_copy(
                        out_hbm.at[0, pl.ds(o, ln)],
                        rows_v.at[b, r, pl.ds(o, ln)],
                        gsem,
                    ).wait()

        def issue_out(i, b):
            pltpu.async_copy(
                rows_v.at[b],
                out_hbm.at[pl.ds(base + i * R_CHUNK, R_CHUNK)], osem)

        def wait_out(b):
            pltpu.make_async_copy(
                rows_v.at[b],
                out_hbm.at[pl.ds(0, R_CHUNK)], osem).wait()

        for b in range(NBUF):
            issue_idx(b, b)

        def pair_body(t, carry):
            for b in range(NBUF):
                i = t * NBUF + b

                @pl.when(i >= NBUF)
                def _():
                    wait_out(b)

                wait_idx(b)
                issue_gathers(b)
                wait_gathers(b)
                issue_out(i, b)

                @pl.when(i + NBUF < n_chunks)
                def _():
                    issue_idx(i + NBUF, b)
            return carry

        lax.fori_loop(0, n_chunks // NBUF, pair_body, 0)

        for b in range(NBUF):
            wait_out(b)

    return k(x, emb_weight)


def kernel(x, emb_weight):
    return _gather_impl(x.astype(jnp.int32), emb_weight)
